# Initial kernel scaffold; baseline (speedup 1.0000x reference)
import jax
import jax.numpy as jnp
from jax.experimental import pallas as pl


def kernel(conv_layer1__weight, conv_layer1__bias, conv_layer2__weight, conv_layer2__bias, conv_layer3__weight, conv_layer3__bias, conv_layer4__weight, conv_layer4__bias, conv_layer5__weight, conv_layer5__bias, conv_layer6__weight, conv_layer6__bias, conv_layer7__weight, conv_layer7__bias, conv_layer11__weight, conv_layer11__bias, x_nchw):
    raise NotImplementedError("write your pallas kernel here")


# speedup vs baseline: 1.2200x; 1.2200x over previous
import jax
import jax.numpy as jnp
from jax.experimental import pallas as pl
from jax.experimental.pallas import tpu as pltpu

KSIZE = 5
NTAPS = KSIZE * KSIZE
CMID = 64
NLAYERS = 8
KP = NTAPS * CMID + CMID
BIAS_ROW = NTAPS * CMID
H = 48
W = 48
SPATIAL = H * W
GUARD = 2 * W + 2
CANVAS = SPATIAL + 2 * GUARD


def _kernel_body(x_ref, w_ref, mask_ref, *out_refs):
    actp_ref, col_ref = out_refs[-2], out_refs[-1]
    before_refs = out_refs[0:NLAYERS - 1]
    after_refs = out_refs[NLAYERS - 1:2 * (NLAYERS - 1)]
    final_ref = out_refs[2 * (NLAYERS - 1)]

    actp_ref[...] = jnp.zeros((CMID, CANVAS), jnp.bfloat16)
    actp_ref[0:8, GUARD:GUARD + SPATIAL] = x_ref[0]

    row = jax.lax.broadcasted_iota(jnp.int32, (KP - BIAS_ROW, SPATIAL), 0)
    col_ref[BIAS_ROW:, :] = (row == 0).astype(jnp.bfloat16)

    for l in range(NLAYERS):
        for t in range(NTAPS):
            dy, dx = divmod(t, KSIZE)
            s = GUARD + (dy - 2) * W + (dx - 2)
            src = actp_ref[:, s:s + SPATIAL]
            if dx != 2:
                src = src * mask_ref[dx, :]
            col_ref[t * CMID:(t + 1) * CMID, :] = src

        pre = jnp.dot(w_ref[l], col_ref[...],
                      preferred_element_type=jnp.float32)

        if l < NLAYERS - 1:
            before_refs[l][0] = pre
            relu = jnp.maximum(pre, 0.0)
            after_refs[l][0] = relu
            actp_ref[:, GUARD:GUARD + SPATIAL] = relu.astype(jnp.bfloat16)
        else:
            final_ref[0] = pre[0:8]


def _pack_layer(weight, bias):
    cout, cin = weight.shape[0], weight.shape[1]
    w = jnp.transpose(weight, (0, 2, 3, 1))
    w = jnp.pad(w, ((0, CMID - cout), (0, 0), (0, 0), (0, CMID - cin)))
    w = w.reshape(CMID, NTAPS * CMID)
    b = jnp.pad(bias, (0, CMID - cout)).reshape(CMID, 1)
    tail = jnp.concatenate([b, jnp.zeros((CMID, KP - BIAS_ROW - 1), w.dtype)],
                           axis=1)
    return jnp.concatenate([w, tail], axis=1).astype(jnp.bfloat16)


@jax.jit
def _forward(params, x_nchw):
    N, Cin, _, _ = x_nchw.shape
    names = [f"conv_layer{i}" for i in list(range(1, 8)) + [11]]
    w_all = jnp.stack([_pack_layer(params[n]["weight"], params[n]["bias"])
                       for n in names])

    xp = jnp.pad(x_nchw, ((0, 0), (0, 8 - Cin), (0, 0), (0, 0)))
    x_flat = xp.reshape(N, 8, SPATIAL).astype(jnp.bfloat16)

    xpos = jnp.arange(SPATIAL) % W
    masks = jnp.stack([((xpos + (dx - 2) >= 0) & (xpos + (dx - 2) < W))
                       for dx in range(KSIZE)] + [xpos < 0] * 3)
    masks = masks.astype(jnp.bfloat16)

    out_shapes = (
        [jax.ShapeDtypeStruct((N, CMID, SPATIAL), jnp.float32)] * 14
        + [jax.ShapeDtypeStruct((N, 8, SPATIAL), jnp.float32)]
    )
    out_specs = (
        [pl.BlockSpec((1, CMID, SPATIAL), lambda n: (n, 0, 0))] * 14
        + [pl.BlockSpec((1, 8, SPATIAL), lambda n: (n, 0, 0))]
    )
    outs = pl.pallas_call(
        _kernel_body,
        out_shape=out_shapes,
        grid_spec=pltpu.PrefetchScalarGridSpec(
            num_scalar_prefetch=0,
            grid=(N,),
            in_specs=[
                pl.BlockSpec((1, 8, SPATIAL), lambda n: (n, 0, 0)),
                pl.BlockSpec((NLAYERS, CMID, KP), lambda n: (0, 0, 0)),
                pl.BlockSpec((8, SPATIAL), lambda n: (0, 0)),
            ],
            out_specs=out_specs,
            scratch_shapes=[
                pltpu.VMEM((CMID, CANVAS), jnp.bfloat16),
                pltpu.VMEM((KP, SPATIAL), jnp.bfloat16),
            ]),
        compiler_params=pltpu.CompilerParams(
            dimension_semantics=("parallel",),
            vmem_limit_bytes=64 * 1024 * 1024),
    )(x_flat, w_all, masks)

    mid = {}
    for i in range(7):
        mid[f"out{i + 1}_before"] = outs[i].reshape(N, CMID, H, W)
        mid[f"out{i + 1}_after"] = outs[7 + i].reshape(N, CMID, H, W)
    output = outs[14][:, :2].reshape(N, 2, H, W)
    return output, mid


def kernel(conv_layer1__weight, conv_layer1__bias,
           conv_layer2__weight, conv_layer2__bias,
           conv_layer3__weight, conv_layer3__bias,
           conv_layer4__weight, conv_layer4__bias,
           conv_layer5__weight, conv_layer5__bias,
           conv_layer6__weight, conv_layer6__bias,
           conv_layer7__weight, conv_layer7__bias,
           conv_layer11__weight, conv_layer11__bias,
           x_nchw):
    params = {
        "conv_layer1": {"weight": conv_layer1__weight, "bias": conv_layer1__bias},
        "conv_layer2": {"weight": conv_layer2__weight, "bias": conv_layer2__bias},
        "conv_layer3": {"weight": conv_layer3__weight, "bias": conv_layer3__bias},
        "conv_layer4": {"weight": conv_layer4__weight, "bias": conv_layer4__bias},
        "conv_layer5": {"weight": conv_layer5__weight, "bias": conv_layer5__bias},
        "conv_layer6": {"weight": conv_layer6__weight, "bias": conv_layer6__bias},
        "conv_layer7": {"weight": conv_layer7__weight, "bias": conv_layer7__bias},
        "conv_layer11": {"weight": conv_layer11__weight, "bias": conv_layer11__bias},
    }
    return _forward(params, x_nchw)


# speedup vs baseline: 1.2993x; 1.0650x over previous
import jax
import jax.numpy as jnp
from jax.experimental import pallas as pl
from jax.experimental.pallas import tpu as pltpu

KSIZE = 5
NTAPS = KSIZE * KSIZE
CMID = 64
NLAYERS = 8
KP = NTAPS * CMID + CMID
BIAS_ROW = NTAPS * CMID
H = 48
W = 48
SPATIAL = H * W
GUARD = 2 * W + 2
CANVAS = SPATIAL + 2 * GUARD
K1 = 256
BIAS1_ROW = NTAPS * 8


def _kernel_body(x_ref, w1_ref, w_ref, mask_ref, *out_refs):
    actp_ref, col_ref = out_refs[-2], out_refs[-1]
    before_refs = out_refs[0:NLAYERS - 1]
    after_refs = out_refs[NLAYERS - 1:2 * (NLAYERS - 1)]
    final_ref = out_refs[2 * (NLAYERS - 1)]

    actp_ref[...] = jnp.zeros((CMID, CANVAS), jnp.bfloat16)
    actp_ref[0:8, GUARD:GUARD + SPATIAL] = x_ref[0]

    row = jax.lax.broadcasted_iota(jnp.int32, (KP - BIAS_ROW, SPATIAL), 0)
    col_ref[BIAS_ROW:, :] = (row == 0).astype(jnp.bfloat16)
    row1 = jax.lax.broadcasted_iota(jnp.int32, (K1 - NTAPS * 8, SPATIAL), 0)

    for t in range(NTAPS):
        dy, dx = divmod(t, KSIZE)
        s = GUARD + (dy - 2) * W + (dx - 2)
        src = actp_ref[0:8, s:s + SPATIAL]
        if dx != 2:
            src = src * mask_ref[dx, :]
        col_ref[t * 8:(t + 1) * 8, :] = src
    col_ref[NTAPS * 8:K1, :] = (row1 == 0).astype(jnp.bfloat16)
    pre = jnp.dot(w1_ref[...], col_ref[0:K1, :],
                  preferred_element_type=jnp.float32)
    before_refs[0][0] = pre
    relu = jnp.maximum(pre, 0.0)
    after_refs[0][0] = relu
    actp_ref[:, GUARD:GUARD + SPATIAL] = relu.astype(jnp.bfloat16)

    for l in range(1, NLAYERS):
        for t in range(NTAPS):
            dy, dx = divmod(t, KSIZE)
            s = GUARD + (dy - 2) * W + (dx - 2)
            src = actp_ref[:, s:s + SPATIAL]
            if dx != 2:
                src = src * mask_ref[dx, :]
            col_ref[t * CMID:(t + 1) * CMID, :] = src

        pre = jnp.dot(w_ref[l - 1], col_ref[...],
                      preferred_element_type=jnp.float32)

        if l < NLAYERS - 1:
            before_refs[l][0] = pre
            relu = jnp.maximum(pre, 0.0)
            after_refs[l][0] = relu
            actp_ref[:, GUARD:GUARD + SPATIAL] = relu.astype(jnp.bfloat16)
        else:
            final_ref[0] = pre[0:8]


def _pack_layer1(weight, bias):
    cout, cin = weight.shape[0], weight.shape[1]
    w = jnp.transpose(weight, (0, 2, 3, 1))
    w = jnp.pad(w, ((0, CMID - cout), (0, 0), (0, 0), (0, 8 - cin)))
    w = w.reshape(CMID, NTAPS * 8)
    b = bias.reshape(CMID, 1)
    tail = jnp.concatenate([b, jnp.zeros((CMID, K1 - BIAS1_ROW - 1), w.dtype)],
                           axis=1)
    return jnp.concatenate([w, tail], axis=1).astype(jnp.bfloat16)


def _pack_layer(weight, bias):
    cout, cin = weight.shape[0], weight.shape[1]
    w = jnp.transpose(weight, (0, 2, 3, 1))
    w = jnp.pad(w, ((0, CMID - cout), (0, 0), (0, 0), (0, CMID - cin)))
    w = w.reshape(CMID, NTAPS * CMID)
    b = jnp.pad(bias, (0, CMID - cout)).reshape(CMID, 1)
    tail = jnp.concatenate([b, jnp.zeros((CMID, KP - BIAS_ROW - 1), w.dtype)],
                           axis=1)
    return jnp.concatenate([w, tail], axis=1).astype(jnp.bfloat16)


@jax.jit
def _forward(params, x_nchw):
    N, Cin, _, _ = x_nchw.shape
    names = [f"conv_layer{i}" for i in list(range(2, 8)) + [11]]
    w1 = _pack_layer1(params["conv_layer1"]["weight"],
                      params["conv_layer1"]["bias"])
    w_all = jnp.stack([_pack_layer(params[n]["weight"], params[n]["bias"])
                       for n in names])

    xp = jnp.pad(x_nchw, ((0, 0), (0, 8 - Cin), (0, 0), (0, 0)))
    x_flat = xp.reshape(N, 8, SPATIAL).astype(jnp.bfloat16)

    xpos = jnp.arange(SPATIAL) % W
    masks = jnp.stack([((xpos + (dx - 2) >= 0) & (xpos + (dx - 2) < W))
                       for dx in range(KSIZE)] + [xpos < 0] * 3)
    masks = masks.astype(jnp.bfloat16)

    out_shapes = (
        [jax.ShapeDtypeStruct((N, CMID, SPATIAL), jnp.float32)] * 14
        + [jax.ShapeDtypeStruct((N, 8, SPATIAL), jnp.float32)]
    )
    out_specs = (
        [pl.BlockSpec((1, CMID, SPATIAL), lambda n: (n, 0, 0))] * 14
        + [pl.BlockSpec((1, 8, SPATIAL), lambda n: (n, 0, 0))]
    )
    outs = pl.pallas_call(
        _kernel_body,
        out_shape=out_shapes,
        grid_spec=pltpu.PrefetchScalarGridSpec(
            num_scalar_prefetch=0,
            grid=(N,),
            in_specs=[
                pl.BlockSpec((1, 8, SPATIAL), lambda n: (n, 0, 0)),
                pl.BlockSpec((CMID, K1), lambda n: (0, 0)),
                pl.BlockSpec((NLAYERS - 1, CMID, KP), lambda n: (0, 0, 0)),
                pl.BlockSpec((8, SPATIAL), lambda n: (0, 0)),
            ],
            out_specs=out_specs,
            scratch_shapes=[
                pltpu.VMEM((CMID, CANVAS), jnp.bfloat16),
                pltpu.VMEM((KP, SPATIAL), jnp.bfloat16),
            ]),
        compiler_params=pltpu.CompilerParams(
            dimension_semantics=("parallel",),
            vmem_limit_bytes=64 * 1024 * 1024),
    )(x_flat, w1, w_all, masks)

    mid = {}
    for i in range(7):
        mid[f"out{i + 1}_before"] = outs[i].reshape(N, CMID, H, W)
        mid[f"out{i + 1}_after"] = outs[7 + i].reshape(N, CMID, H, W)
    output = outs[14][:, :2].reshape(N, 2, H, W)
    return output, mid


def kernel(conv_layer1__weight, conv_layer1__bias,
           conv_layer2__weight, conv_layer2__bias,
           conv_layer3__weight, conv_layer3__bias,
           conv_layer4__weight, conv_layer4__bias,
           conv_layer5__weight, conv_layer5__bias,
           conv_layer6__weight, conv_layer6__bias,
           conv_layer7__weight, conv_layer7__bias,
           conv_layer11__weight, conv_layer11__bias,
           x_nchw):
    params = {
        "conv_layer1": {"weight": conv_layer1__weight, "bias": conv_layer1__bias},
        "conv_layer2": {"weight": conv_layer2__weight, "bias": conv_layer2__bias},
        "conv_layer3": {"weight": conv_layer3__weight, "bias": conv_layer3__bias},
        "conv_layer4": {"weight": conv_layer4__weight, "bias": conv_layer4__bias},
        "conv_layer5": {"weight": conv_layer5__weight, "bias": conv_layer5__bias},
        "conv_layer6": {"weight": conv_layer6__weight, "bias": conv_layer6__bias},
        "conv_layer7": {"weight": conv_layer7__weight, "bias": conv_layer7__bias},
        "conv_layer11": {"weight": conv_layer11__weight, "bias": conv_layer11__bias},
    }
    return _forward(params, x_nchw)
